# baseline (speedup 1.0000x reference)
import functools

import numpy as np
import jax
import jax.numpy as jnp
from jax import lax
from jax.experimental import pallas as pl
from jax.experimental.pallas import tpu as pltpu
from jax.experimental.pallas import tpu_sc as plsc

EMBED = 32
MAX_LEN = 202
SEQ = 200
BATCH = 4096
N_TOK = BATCH * SEQ
NUM_WORKERS = 32
PER_W = N_TOK // NUM_WORKERS
CHUNK = 512
N_CHUNKS = PER_W // CHUNK
KSUB = CHUNK // 128
IDX_ROWS = N_TOK // 128


def _make_pe() -> jnp.ndarray:
    pos = np.arange(MAX_LEN, dtype=np.float32)[:, None]
    div = np.exp(np.arange(0, EMBED, 2, dtype=np.float32) * -(np.log(10000.0) / EMBED))
    pe = np.zeros((MAX_LEN, EMBED), dtype=np.float32)
    pe[:, 0::2] = np.sin(pos * div)
    pe[:, 1::2] = np.cos(pos * div)
    return jnp.asarray(pe[:SEQ])


_MESH = plsc.VectorSubcoreMesh(core_axis_name="c", subcore_axis_name="s")


@functools.partial(
    pl.kernel,
    out_type=jax.ShapeDtypeStruct((N_TOK, EMBED), jnp.float32),
    mesh=_MESH,
    compiler_params=pltpu.CompilerParams(use_tc_tiling_on_sc=False),
    scratch_types=[
        pltpu.VMEM((KSUB, 128), jnp.int32),
        pltpu.VMEM((KSUB, 128), jnp.int32),
        pltpu.VMEM((KSUB, 128), jnp.int32),
        pltpu.VMEM((KSUB, 128), jnp.int32),
        pltpu.VMEM((CHUNK, EMBED), jnp.float32),
        pltpu.VMEM((CHUNK, EMBED), jnp.float32),
        pltpu.VMEM((CHUNK, EMBED), jnp.float32),
        pltpu.VMEM((CHUNK, EMBED), jnp.float32),
        pltpu.VMEM((SEQ, EMBED), jnp.float32),
        pltpu.SemaphoreType.DMA,
        pltpu.SemaphoreType.DMA,
    ],
)
def _emb_kernel(grid_tab, time_tab, event_tab, hand_tab, pe_tab,
                gidx, tidx, eidx, hidx, out,
                s_gi, s_ti, s_ei, s_hi, r_g, r_t, r_e, r_h, pe_v,
                sem_i, sem_g):
    wid = lax.axis_index("s") * 2 + lax.axis_index("c")
    pltpu.sync_copy(pe_tab, pe_v)
    idx_row0 = wid * (PER_W // 128)

    def chunk_body(i, carry):
        rbase = idx_row0 + i * KSUB
        cps = [
            pltpu.async_copy(gidx.at[pl.ds(rbase, KSUB)], s_gi, sem_i),
            pltpu.async_copy(tidx.at[pl.ds(rbase, KSUB)], s_ti, sem_i),
            pltpu.async_copy(eidx.at[pl.ds(rbase, KSUB)], s_ei, sem_i),
            pltpu.async_copy(hidx.at[pl.ds(rbase, KSUB)], s_hi, sem_i),
        ]
        for cp in cps:
            cp.wait()
        gcps = []
        for k in range(KSUB):
            dst = pl.ds(k * 128, 128)
            gcps.append(pltpu.async_copy(grid_tab.at[s_gi.at[k]], r_g.at[dst], sem_g))
            gcps.append(pltpu.async_copy(time_tab.at[s_ti.at[k]], r_t.at[dst], sem_g))
            gcps.append(pltpu.async_copy(event_tab.at[s_ei.at[k]], r_e.at[dst], sem_g))
            gcps.append(pltpu.async_copy(hand_tab.at[s_hi.at[k]], r_h.at[dst], sem_g))
        for cp in gcps:
            cp.wait()

        base_tok = wid * PER_W + i * CHUNK
        pe0 = lax.rem(base_tok, SEQ)

        def row_body(r, c2):
            pr = lax.rem(pe0 + r, SEQ)
            for h0 in (0, 16):
                acc = (r_g[r, h0:h0 + 16] + r_t[r, h0:h0 + 16]
                       + r_e[r, h0:h0 + 16] + r_h[r, h0:h0 + 16]
                       + pe_v[pr, h0:h0 + 16])
                r_g[r, h0:h0 + 16] = acc
            return c2

        lax.fori_loop(0, CHUNK, row_body, 0)
        pltpu.sync_copy(r_g, out.at[pl.ds(base_tok, CHUNK)])
        return carry

    lax.fori_loop(0, N_CHUNKS, chunk_body, 0)


def kernel(grid, timestamp, event, hand, grid_table, time_table, event_table,
           hand_table, train_mode):
    pe = _make_pe()
    gi = grid.astype(jnp.int32).reshape(IDX_ROWS, 128)
    ti = timestamp.astype(jnp.int32).reshape(IDX_ROWS, 128)
    ei = event.astype(jnp.int32).reshape(IDX_ROWS, 128)
    hi = hand.astype(jnp.int32).reshape(IDX_ROWS, 128)
    out = _emb_kernel(grid_table, time_table, event_table, hand_table, pe,
                      gi, ti, ei, hi)
    return out.reshape(BATCH, SEQ, EMBED)
